# baseline (speedup 1.0000x reference)
import numpy as np
import jax
import jax.numpy as jnp
from jax import lax
from jax.experimental import pallas as pl
from jax.experimental.pallas import tpu as pltpu
from jax.experimental.pallas import tpu_sc as plsc

_N = 10000
_F = 128
_E = 160000
_SPH = 15
_H = 4
_DH = _F // _H
_AVG = 16.0

_NW = 32
_CHUNK = 64
_GCHUNK = 64
_EPW = 1280
_E_HALF = _NW * _EPW
_E_PAD = 4 * _E_HALF
_N_PAD = 10240
_ROWS = _N_PAD // 16

_BE = 1024
_BN = 1024

_DEGS = (3, 5, 7)


def _build_consts():
    g = np.zeros((16, 16), np.float32)
    m = np.zeros((16, 16), np.float32)
    o = 0
    for i, d in enumerate(_DEGS):
        g[o:o + d, i] = 1.0
        m[i, o:o + d] = 1.0
        o += d
    hm = np.zeros((_F, _F), np.float32)
    for h in range(_H):
        hm[h * _DH:(h + 1) * _DH, h * _DH:(h + 1) * _DH] = 1.0
    return g, m, m[:8], hm


_G16, _M16, _M8, _HMASK = _build_consts()


def _bf16_top(x):
    xi = jax.lax.bitcast_convert_type(x, jnp.int32)
    rnd = jnp.int32(0x7FFF) + \
        jnp.bitwise_and(jax.lax.shift_right_logical(xi, 16), jnp.int32(1))
    return jax.lax.shift_right_logical(xi + rnd, 16)


def _pack_bf16(x):
    n = x.shape[1] // 2
    hi = jax.lax.shift_left(_bf16_top(x[:, :n]), 16)
    lo = _bf16_top(x[:, n:])
    return jax.lax.bitcast_convert_type(jnp.bitwise_or(hi, lo), jnp.float32)


def _unpack_bf16(x):
    xi = jax.lax.bitcast_convert_type(x, jnp.int32)
    a = jnp.bitwise_and(xi, jnp.int32(-65536))
    b = jax.lax.shift_left(xi, 16)
    return jnp.concatenate([
        jax.lax.bitcast_convert_type(a, jnp.float32),
        jax.lax.bitcast_convert_type(b, jnp.float32)], axis=1)


def _proj_body(nf, chi, wq, wkv, qc_out, kvc_out):
    x = nf[...]
    ch = chi[...]
    hi = ch.astype(jnp.bfloat16).astype(jnp.float32)
    lo = ch - hi
    chrow = _pack_bf16(jnp.concatenate([hi, lo], axis=1))
    b = x.shape[0]
    qc_out[...] = jnp.concatenate(
        [_pack_bf16(x @ wq[...]), chrow, jnp.zeros((b, 48), jnp.float32)],
        axis=1)
    kvc_out[...] = jnp.concatenate(
        [_pack_bf16(x @ wkv[...]), chrow, jnp.zeros((b, 112), jnp.float32)],
        axis=1)


def _edge_body(ef, shp, qcr, kvcs,
               wfbr0, bfbr0, wfbr1, bfbr1, wfbs0, bfbs0, wfbs1, bfbs1,
               wgbr0, bgbr0, wgbr1, bgbr1, wgbs0, bgbs0, wgbs1, bgbs1,
               gmat, mmat, hmask, msg_out, gv_out):
    qc = qcr[...]
    kvc = kvcs[...]
    q = _unpack_bf16(qc[:, :64])
    chu_r = _unpack_bf16(qc[:, 64:80])
    kvu = _unpack_bf16(kvc[:, :128])
    chu_s = _unpack_bf16(kvc[:, 128:144])
    d = (chu_s[:, :16] + chu_s[:, 16:]) - (chu_r[:, :16] + chu_r[:, 16:])
    cs16 = (d * d) @ gmat[...]
    e = ef[...]
    h1 = jax.nn.silu(e @ wfbr0[...] + bfbr0[...])
    h2 = jax.nn.silu(cs16 @ wfbs0[...] + bfbs0[...])
    wfil = h1 @ wfbr1[...] + bfbr1[...] + h2 @ wfbs1[...] + bfbs1[...]
    sh = shp[...]
    cut = sh[:, 15:16]
    t = q * wfil * kvu[:, :_F]
    a = (t @ hmask[...]) * (cut * np.float32(1.0 / (np.sqrt(_DH) * _AVG)))
    msg_out[...] = a * kvu[:, _F:]
    g1 = jax.nn.silu(e @ wgbr0[...] + bgbr0[...])
    g2 = jax.nn.silu(cs16 @ wgbs0[...] + bgbs0[...])
    c16 = (g1 @ wgbr1[...] + bgbr1[...] + g2 @ wgbs1[...] + bgbs1[...]) \
        * (cut * np.float32(1.0 / _AVG))
    gv16 = (c16 @ mmat[...]) * sh
    gv_out[...] = jnp.concatenate(
        [gv16, jnp.zeros((gv16.shape[0], 112), jnp.float32)], axis=1)


def _fin_body(nf, chi, pm0, pm1, pm2, pm3, pg0, pg1, pg2, pg3,
              wi0, bi0, wi1, bi1, gmat, m8, outn, outc):
    node2 = nf[...] + (pm0[0] + pm0[1]) + (pm1[0] + pm1[1]) \
        + (pm2[0] + pm2[1]) + (pm3[0] + pm3[1])
    chi2 = chi[...] + (pg0[0, :, :16] + pg0[1, :, :16]) \
        + (pg1[0, :, :16] + pg1[1, :, :16]) \
        + (pg2[0, :, :16] + pg2[1, :, :16]) \
        + (pg3[0, :, :16] + pg3[1, :, :16])
    cn16 = (chi2 * chi2) @ gmat[...]
    y = jnp.concatenate([node2, cn16[:, :8]], axis=1)
    h = jax.nn.silu(y @ wi0[...] + bi0[...])
    dd = h @ wi1[...] + bi1[...]
    outn[...] = node2 + dd[:, :_F]
    outc[...] = chi2 + (dd[:, _F:] @ m8[...]) * chi2


def _gather_body(qtab, kvtab, snd, rcv, qr_out, kvs_out,
                 i0s, i0r, i1s, i1r, q0, q1, kv0, kv1,
                 si0, si1, sg0, sg1, sw0, sw1):
    wid = lax.axis_index("s") * 2 + lax.axis_index("c")
    base = wid * _EPW
    idx_s = (i0s, i1s)
    idx_r = (i0r, i1r)
    bq = (q0, q1)
    bkv = (kv0, kv1)
    si = (si0, si1)
    sg = (sg0, sg1)
    sw = (sw0, sw1)

    def issue_idx(ch, b):
        o = base + ch * _GCHUNK
        pltpu.async_copy(snd.at[pl.ds(o, _GCHUNK)], idx_s[b], si[b])
        pltpu.async_copy(rcv.at[pl.ds(o, _GCHUNK)], idx_r[b], si[b])

    def wait_idx(ch, b):
        o = base + ch * _GCHUNK
        pltpu.make_async_copy(snd.at[pl.ds(o, _GCHUNK)], idx_s[b], si[b]).wait()
        pltpu.make_async_copy(rcv.at[pl.ds(o, _GCHUNK)], idx_r[b], si[b]).wait()

    def issue_g(b):
        pltpu.async_copy(qtab.at[idx_r[b]], bq[b], sg[b])
        pltpu.async_copy(kvtab.at[idx_s[b]], bkv[b], sg[b])

    def wait_g(b):
        pltpu.make_async_copy(qtab.at[idx_r[b]], bq[b], sg[b]).wait()
        pltpu.make_async_copy(kvtab.at[idx_s[b]], bkv[b], sg[b]).wait()

    def issue_wb(ch, b):
        o = base + ch * _GCHUNK
        pltpu.async_copy(bq[b], qr_out.at[pl.ds(o, _GCHUNK)], sw[b])
        pltpu.async_copy(bkv[b], kvs_out.at[pl.ds(o, _GCHUNK)], sw[b])

    def wait_wb(ch, b):
        o = base + ch * _GCHUNK
        pltpu.make_async_copy(bq[b], qr_out.at[pl.ds(o, _GCHUNK)], sw[b]).wait()
        pltpu.make_async_copy(bkv[b], kvs_out.at[pl.ds(o, _GCHUNK)], sw[b]).wait()

    issue_idx(0, 0)
    wait_idx(0, 0)
    issue_g(0)
    issue_idx(1, 1)

    def body(g, carry):
        wait_g(0)
        issue_wb(2 * g, 0)

        wait_idx(2 * g + 1, 1)

        @pl.when(g > 0)
        def _():
            wait_wb(2 * g - 1, 1)

        issue_g(1)
        issue_idx(2 * g + 2, 0)
        wait_g(1)
        issue_wb(2 * g + 1, 1)
        wait_idx(2 * g + 2, 0)
        wait_wb(2 * g, 0)
        issue_g(0)
        issue_idx(2 * g + 3, 1)
        return carry

    nch = _EPW // _GCHUNK
    lax.fori_loop(0, nch // 2 - 1, body, 0)
    wait_g(0)
    issue_wb(nch - 2, 0)
    wait_idx(nch - 1, 1)
    wait_wb(nch - 3, 1)
    issue_g(1)
    wait_g(1)
    issue_wb(nch - 1, 1)
    wait_wb(nch - 2, 0)
    wait_wb(nch - 1, 1)


def _scatter_body(rcv, vals, zeros, part_out, acc,
                  i0, i1, v0, v1, sl0, sl1, ss0, ss1):
    cid = lax.axis_index("c")
    sid = lax.axis_index("s")
    wid = sid * 2 + cid
    r0 = sid * _ROWS
    pltpu.sync_copy(zeros.at[pl.ds(r0, _ROWS)], acc.at[pl.ds(r0, _ROWS)])
    plsc.subcore_barrier()
    base = wid * _EPW
    idx = (i0, i1)
    bv = (v0, v1)
    sl = (sl0, sl1)
    ss = (ss0, ss1)

    def issue_ld(ch, b):
        o = base + ch * _CHUNK
        pltpu.async_copy(rcv.at[pl.ds(o, _CHUNK)], idx[b], sl[b])
        pltpu.async_copy(vals.at[pl.ds(o, _CHUNK)], bv[b], sl[b])

    def wait_ld(ch, b):
        o = base + ch * _CHUNK
        pltpu.make_async_copy(rcv.at[pl.ds(o, _CHUNK)], idx[b], sl[b]).wait()
        pltpu.make_async_copy(vals.at[pl.ds(o, _CHUNK)], bv[b], sl[b]).wait()

    def issue_sc(b):
        pltpu.async_copy(bv[b], acc.at[idx[b]], ss[b], add=True)

    def wait_sc(b):
        pltpu.make_async_copy(bv[b], acc.at[idx[b]], ss[b]).wait()

    issue_ld(0, 0)

    def body(g, carry):
        wait_ld(2 * g, 0)
        issue_sc(0)

        @pl.when(g > 0)
        def _():
            wait_sc(1)

        issue_ld(2 * g + 1, 1)
        wait_ld(2 * g + 1, 1)
        issue_sc(1)
        wait_sc(0)

        @pl.when(g < _EPW // _CHUNK // 2 - 1)
        def _():
            issue_ld(2 * g + 2, 0)

        return carry

    lax.fori_loop(0, _EPW // _CHUNK // 2, body, 0)
    wait_sc(1)
    plsc.subcore_barrier()
    pltpu.sync_copy(acc.at[pl.ds(r0, _ROWS)], part_out.at[cid, pl.ds(r0, _ROWS)])


def _full(shape):
    return pl.BlockSpec(shape, lambda i: tuple(0 for _ in shape))


_proj_call = pl.pallas_call(
    _proj_body,
    grid=(_N_PAD // _BN,),
    in_specs=[
        pl.BlockSpec((_BN, _F), lambda i: (i, 0)),
        pl.BlockSpec((_BN, 16), lambda i: (i, 0)),
        _full((_F, _F)),
        _full((_F, 2 * _F)),
    ],
    out_specs=[
        pl.BlockSpec((_BN, _F), lambda i: (i, 0)),
        pl.BlockSpec((_BN, 2 * _F), lambda i: (i, 0)),
    ],
    out_shape=[
        jax.ShapeDtypeStruct((_N_PAD, _F), jnp.float32),
        jax.ShapeDtypeStruct((_N_PAD, 2 * _F), jnp.float32),
    ],
)

_edge_call = pl.pallas_call(
    _edge_body,
    grid=(_E_HALF // _BE,),
    in_specs=[
        pl.BlockSpec((_BE, 16), lambda i: (i, 0)),
        pl.BlockSpec((_BE, 16), lambda i: (i, 0)),
        pl.BlockSpec((_BE, _F), lambda i: (i, 0)),
        pl.BlockSpec((_BE, 2 * _F), lambda i: (i, 0)),
        _full((16, _F)), _full((1, _F)), _full((_F, _F)), _full((1, _F)),
        _full((16, _F)), _full((1, _F)), _full((_F, _F)), _full((1, _F)),
        _full((16, _F)), _full((1, _F)), _full((_F, 16)), _full((1, 16)),
        _full((16, _F)), _full((1, _F)), _full((_F, 16)), _full((1, 16)),
        _full((16, 16)), _full((16, 16)), _full((_F, _F)),
    ],
    out_specs=[
        pl.BlockSpec((_BE, _F), lambda i: (i, 0)),
        pl.BlockSpec((_BE, _F), lambda i: (i, 0)),
    ],
    out_shape=[
        jax.ShapeDtypeStruct((_E_HALF, _F), jnp.float32),
        jax.ShapeDtypeStruct((_E_HALF, _F), jnp.float32),
    ],
)

_fin_call = pl.pallas_call(
    _fin_body,
    grid=(_N_PAD // _BN,),
    in_specs=[
        pl.BlockSpec((_BN, _F), lambda i: (i, 0)),
        pl.BlockSpec((_BN, 16), lambda i: (i, 0)),
        pl.BlockSpec((2, _BN, _F), lambda i: (0, i, 0)),
        pl.BlockSpec((2, _BN, _F), lambda i: (0, i, 0)),
        pl.BlockSpec((2, _BN, _F), lambda i: (0, i, 0)),
        pl.BlockSpec((2, _BN, _F), lambda i: (0, i, 0)),
        pl.BlockSpec((2, _BN, _F), lambda i: (0, i, 0)),
        pl.BlockSpec((2, _BN, _F), lambda i: (0, i, 0)),
        pl.BlockSpec((2, _BN, _F), lambda i: (0, i, 0)),
        pl.BlockSpec((2, _BN, _F), lambda i: (0, i, 0)),
        _full((136, _F)), _full((1, _F)), _full((_F, 136)), _full((1, 136)),
        _full((16, 16)), _full((8, 16)),
    ],
    out_specs=[
        pl.BlockSpec((_BN, _F), lambda i: (i, 0)),
        pl.BlockSpec((_BN, 16), lambda i: (i, 0)),
    ],
    out_shape=[
        jax.ShapeDtypeStruct((_N_PAD, _F), jnp.float32),
        jax.ShapeDtypeStruct((_N_PAD, 16), jnp.float32),
    ],
)

_sc_calls_cache = []


def _sc_calls():
    if _sc_calls_cache:
        return _sc_calls_cache[0]
    mesh = plsc.VectorSubcoreMesh(core_axis_name="c", subcore_axis_name="s")
    gather_call = pl.kernel(
        _gather_body,
        out_type=[
            jax.ShapeDtypeStruct((_E_HALF, _F), jnp.float32),
            jax.ShapeDtypeStruct((_E_HALF, 2 * _F), jnp.float32),
        ],
        mesh=mesh,
        scratch_types=[
            pltpu.VMEM((_GCHUNK,), jnp.int32),
            pltpu.VMEM((_GCHUNK,), jnp.int32),
            pltpu.VMEM((_GCHUNK,), jnp.int32),
            pltpu.VMEM((_GCHUNK,), jnp.int32),
            pltpu.VMEM((_GCHUNK, _F), jnp.float32),
            pltpu.VMEM((_GCHUNK, _F), jnp.float32),
            pltpu.VMEM((_GCHUNK, 2 * _F), jnp.float32),
            pltpu.VMEM((_GCHUNK, 2 * _F), jnp.float32),
            pltpu.SemaphoreType.DMA,
            pltpu.SemaphoreType.DMA,
            pltpu.SemaphoreType.DMA,
            pltpu.SemaphoreType.DMA,
            pltpu.SemaphoreType.DMA,
            pltpu.SemaphoreType.DMA,
        ],
    )
    scatter_call = pl.kernel(
        _scatter_body,
        out_type=jax.ShapeDtypeStruct((2, _N_PAD, _F), jnp.float32),
        mesh=mesh,
        scratch_types=[
            pltpu.VMEM_SHARED((_N_PAD, _F), jnp.float32),
            pltpu.VMEM((_CHUNK,), jnp.int32),
            pltpu.VMEM((_CHUNK,), jnp.int32),
            pltpu.VMEM((_CHUNK, _F), jnp.float32),
            pltpu.VMEM((_CHUNK, _F), jnp.float32),
            pltpu.SemaphoreType.DMA,
            pltpu.SemaphoreType.DMA,
            pltpu.SemaphoreType.DMA,
            pltpu.SemaphoreType.DMA,
        ],
    )
    _sc_calls_cache.append((gather_call, scatter_call))
    return _sc_calls_cache[0]


def kernel(node_feats, chi, edge_feats, edge_sh, cutoffs, senders, receivers,
           W_fbr0, b_fbr0, W_fbr1, b_fbr1, W_fbs0, b_fbs0, W_fbs1, b_fbs1,
           Wq, Wk, Wv,
           W_gbr0, b_gbr0, W_gbr1, b_gbr1, W_gbs0, b_gbs0, W_gbs1, b_gbs1,
           Wi0, bi0, Wi1, bi1):
    f32 = jnp.float32

    nf_pad = jnp.pad(node_feats, ((0, _N_PAD - _N), (0, 0)))
    chi_pad = jnp.pad(chi, ((0, _N_PAD - _N), (0, 1)))
    ef_pad = jnp.pad(edge_feats, ((0, _E_PAD - _E), (0, 0)))
    shp = jnp.pad(jnp.concatenate([edge_sh, cutoffs[:, None]], axis=1),
                  ((0, _E_PAD - _E), (0, 0)))
    snd = jnp.pad(senders, (0, _E_PAD - _E))
    rcv = jnp.pad(receivers, (0, _E_PAD - _E))

    wkv = jnp.concatenate([Wk, Wv], axis=1)
    wfbs0 = jnp.zeros((16, _F), f32).at[:3].set(W_fbs0)
    wgbs0 = jnp.zeros((16, _F), f32).at[:3].set(W_gbs0)
    wgbr1 = jnp.zeros((_F, 16), f32).at[:, :3].set(W_gbr1)
    wgbs1 = jnp.zeros((_F, 16), f32).at[:, :3].set(W_gbs1)
    bgbr1 = jnp.zeros((16,), f32).at[:3].set(b_gbr1)
    bgbs1 = jnp.zeros((16,), f32).at[:3].set(b_gbs1)
    wi0 = jnp.zeros((136, _F), f32).at[:131].set(Wi0)
    wi1 = jnp.zeros((_F, 136), f32).at[:, :131].set(Wi1)
    bi1p = jnp.zeros((136,), f32).at[:131].set(bi1)

    g16 = jnp.asarray(_G16)
    m16 = jnp.asarray(_M16)
    m8 = jnp.asarray(_M8)
    hmask = jnp.asarray(_HMASK)

    gather_call, scatter_call = _sc_calls()
    qc, kvc = _proj_call(nf_pad, chi_pad, Wq, wkv)
    zeros = jnp.zeros((_N_PAD, _F), f32)

    def edge_half(lo, qcr, kvcs):
        return _edge_call(
            lax.dynamic_slice_in_dim(ef_pad, lo, _E_HALF),
            lax.dynamic_slice_in_dim(shp, lo, _E_HALF),
            qcr, kvcs,
            W_fbr0, b_fbr0[None], W_fbr1, b_fbr1[None],
            wfbs0, b_fbs0[None], W_fbs1, b_fbs1[None],
            W_gbr0, b_gbr0[None], wgbr1, bgbr1[None],
            wgbs0, b_gbs0[None], wgbs1, bgbs1[None],
            g16, m16, hmask)

    los = [q * _E_HALF for q in range(4)]
    rcvs = [lax.dynamic_slice_in_dim(rcv, lo, _E_HALF) for lo in los]
    snds = [lax.dynamic_slice_in_dim(snd, lo, _E_HALF) for lo in los]
    gathered = [gather_call(qc, kvc, snds[q], rcvs[q]) for q in range(4)]
    edged = [edge_half(los[q], *gathered[q]) for q in range(4)]
    pms = [scatter_call(rcvs[q], edged[q][0], zeros) for q in range(4)]
    pgs = [scatter_call(rcvs[q], edged[q][1], zeros) for q in range(4)]
    outn, outc = _fin_call(nf_pad, chi_pad, *pms, *pgs,
                           wi0, bi0[None], wi1, bi1p[None], g16, m8)
    return outn[:_N], outc[:_N, :_SPH]


# speedup vs baseline: 1.0175x; 1.0175x over previous
import numpy as np
import jax
import jax.numpy as jnp
from jax import lax
from jax.experimental import pallas as pl
from jax.experimental.pallas import tpu as pltpu
from jax.experimental.pallas import tpu_sc as plsc

_N = 10000
_F = 128
_E = 160000
_SPH = 15
_H = 4
_DH = _F // _H
_AVG = 16.0

_NW = 32
_CHUNK = 64
_GCHUNK = 64
_EPW = 2560
_E_HALF = _NW * _EPW
_E_PAD = 2 * _E_HALF
_N_PAD = 10240
_ROWS = _N_PAD // 16

_BE = 1024
_BN = 1024

_DEGS = (3, 5, 7)


def _build_consts():
    g = np.zeros((16, 16), np.float32)
    m = np.zeros((16, 16), np.float32)
    o = 0
    for i, d in enumerate(_DEGS):
        g[o:o + d, i] = 1.0
        m[i, o:o + d] = 1.0
        o += d
    hm = np.zeros((_F, _F), np.float32)
    for h in range(_H):
        hm[h * _DH:(h + 1) * _DH, h * _DH:(h + 1) * _DH] = 1.0
    return g, m, m[:8], hm


_G16, _M16, _M8, _HMASK = _build_consts()


def _bf16_top(x):
    xi = jax.lax.bitcast_convert_type(x, jnp.int32)
    rnd = jnp.int32(0x7FFF) + \
        jnp.bitwise_and(jax.lax.shift_right_logical(xi, 16), jnp.int32(1))
    return jax.lax.shift_right_logical(xi + rnd, 16)


def _pack_bf16(x):
    n = x.shape[1] // 2
    hi = jax.lax.shift_left(_bf16_top(x[:, :n]), 16)
    lo = _bf16_top(x[:, n:])
    return jax.lax.bitcast_convert_type(jnp.bitwise_or(hi, lo), jnp.float32)


def _unpack_bf16(x):
    xi = jax.lax.bitcast_convert_type(x, jnp.int32)
    a = jnp.bitwise_and(xi, jnp.int32(-65536))
    b = jax.lax.shift_left(xi, 16)
    return jnp.concatenate([
        jax.lax.bitcast_convert_type(a, jnp.float32),
        jax.lax.bitcast_convert_type(b, jnp.float32)], axis=1)


def _proj_body(nf, chi, wq, wkv, qc_out, kvc_out):
    x = nf[...]
    ch = chi[...]
    hi = ch.astype(jnp.bfloat16).astype(jnp.float32)
    lo = ch - hi
    chrow = _pack_bf16(jnp.concatenate([hi, lo], axis=1))
    b = x.shape[0]
    qc_out[...] = jnp.concatenate(
        [_pack_bf16(x @ wq[...]), chrow, jnp.zeros((b, 48), jnp.float32)],
        axis=1)
    kvc_out[...] = jnp.concatenate(
        [_pack_bf16(x @ wkv[...]), chrow, jnp.zeros((b, 112), jnp.float32)],
        axis=1)


def _edge_body(ef, shp, qcr, kvcs,
               wfbr0, bfbr0, wfbr1, bfbr1, wfbs0, bfbs0, wfbs1, bfbs1,
               wgbr0, bgbr0, wgbr1, bgbr1, wgbs0, bgbs0, wgbs1, bgbs1,
               gmat, mmat, hmask, msg_out, gv_out):
    qc = qcr[...]
    kvc = kvcs[...]
    q = _unpack_bf16(qc[:, :64])
    chu_r = _unpack_bf16(qc[:, 64:80])
    kvu = _unpack_bf16(kvc[:, :128])
    chu_s = _unpack_bf16(kvc[:, 128:144])
    d = (chu_s[:, :16] + chu_s[:, 16:]) - (chu_r[:, :16] + chu_r[:, 16:])
    cs16 = (d * d) @ gmat[...]
    e = ef[...]
    h1 = jax.nn.silu(e @ wfbr0[...] + bfbr0[...])
    h2 = jax.nn.silu(cs16 @ wfbs0[...] + bfbs0[...])
    wfil = h1 @ wfbr1[...] + bfbr1[...] + h2 @ wfbs1[...] + bfbs1[...]
    sh = shp[...]
    cut = sh[:, 15:16]
    t = q * wfil * kvu[:, :_F]
    a = (t @ hmask[...]) * (cut * np.float32(1.0 / (np.sqrt(_DH) * _AVG)))
    msg_out[...] = a * kvu[:, _F:]
    g1 = jax.nn.silu(e @ wgbr0[...] + bgbr0[...])
    g2 = jax.nn.silu(cs16 @ wgbs0[...] + bgbs0[...])
    c16 = (g1 @ wgbr1[...] + bgbr1[...] + g2 @ wgbs1[...] + bgbs1[...]) \
        * (cut * np.float32(1.0 / _AVG))
    gv16 = (c16 @ mmat[...]) * sh
    gv_out[...] = jnp.concatenate(
        [gv16, jnp.zeros((gv16.shape[0], 112), jnp.float32)], axis=1)


def _fin_body(nf, chi, pm0, pm1, pg0, pg1, wi0, bi0, wi1, bi1, gmat, m8,
              outn, outc):
    node2 = nf[...] + pm0[0] + pm0[1] + pm1[0] + pm1[1]
    chi2 = chi[...] + pg0[0, :, :16] + pg0[1, :, :16] \
        + pg1[0, :, :16] + pg1[1, :, :16]
    cn16 = (chi2 * chi2) @ gmat[...]
    y = jnp.concatenate([node2, cn16[:, :8]], axis=1)
    h = jax.nn.silu(y @ wi0[...] + bi0[...])
    dd = h @ wi1[...] + bi1[...]
    outn[...] = node2 + dd[:, :_F]
    outc[...] = chi2 + (dd[:, _F:] @ m8[...]) * chi2


def _gather_body(qtab, kvtab, snd, rcv, qr_out, kvs_out,
                 i0s, i0r, i1s, i1r, q0, q1, kv0, kv1,
                 si0, si1, sg0, sg1, sw0, sw1):
    wid = lax.axis_index("s") * 2 + lax.axis_index("c")
    base = wid * _EPW
    idx_s = (i0s, i1s)
    idx_r = (i0r, i1r)
    bq = (q0, q1)
    bkv = (kv0, kv1)
    si = (si0, si1)
    sg = (sg0, sg1)
    sw = (sw0, sw1)

    def issue_idx(ch, b):
        o = base + ch * _GCHUNK
        pltpu.async_copy(snd.at[pl.ds(o, _GCHUNK)], idx_s[b], si[b])
        pltpu.async_copy(rcv.at[pl.ds(o, _GCHUNK)], idx_r[b], si[b])

    def wait_idx(ch, b):
        o = base + ch * _GCHUNK
        pltpu.make_async_copy(snd.at[pl.ds(o, _GCHUNK)], idx_s[b], si[b]).wait()
        pltpu.make_async_copy(rcv.at[pl.ds(o, _GCHUNK)], idx_r[b], si[b]).wait()

    def issue_g(b):
        pltpu.async_copy(qtab.at[idx_r[b]], bq[b], sg[b])
        pltpu.async_copy(kvtab.at[idx_s[b]], bkv[b], sg[b])

    def wait_g(b):
        pltpu.make_async_copy(qtab.at[idx_r[b]], bq[b], sg[b]).wait()
        pltpu.make_async_copy(kvtab.at[idx_s[b]], bkv[b], sg[b]).wait()

    def issue_wb(ch, b):
        o = base + ch * _GCHUNK
        pltpu.async_copy(bq[b], qr_out.at[pl.ds(o, _GCHUNK)], sw[b])
        pltpu.async_copy(bkv[b], kvs_out.at[pl.ds(o, _GCHUNK)], sw[b])

    def wait_wb(ch, b):
        o = base + ch * _GCHUNK
        pltpu.make_async_copy(bq[b], qr_out.at[pl.ds(o, _GCHUNK)], sw[b]).wait()
        pltpu.make_async_copy(bkv[b], kvs_out.at[pl.ds(o, _GCHUNK)], sw[b]).wait()

    issue_idx(0, 0)
    wait_idx(0, 0)
    issue_g(0)
    issue_idx(1, 1)

    def body(g, carry):
        wait_g(0)
        issue_wb(2 * g, 0)

        wait_idx(2 * g + 1, 1)

        @pl.when(g > 0)
        def _():
            wait_wb(2 * g - 1, 1)

        issue_g(1)
        issue_idx(2 * g + 2, 0)
        wait_g(1)
        issue_wb(2 * g + 1, 1)
        wait_idx(2 * g + 2, 0)
        wait_wb(2 * g, 0)
        issue_g(0)
        issue_idx(2 * g + 3, 1)
        return carry

    nch = _EPW // _GCHUNK
    lax.fori_loop(0, nch // 2 - 1, body, 0)
    wait_g(0)
    issue_wb(nch - 2, 0)
    wait_idx(nch - 1, 1)
    wait_wb(nch - 3, 1)
    issue_g(1)
    wait_g(1)
    issue_wb(nch - 1, 1)
    wait_wb(nch - 2, 0)
    wait_wb(nch - 1, 1)


def _scatter_body(rcv, vals, zeros, part_out, acc,
                  i0, i1, v0, v1, sl0, sl1, ss0, ss1):
    cid = lax.axis_index("c")
    sid = lax.axis_index("s")
    wid = sid * 2 + cid
    r0 = sid * _ROWS
    pltpu.sync_copy(zeros.at[pl.ds(r0, _ROWS)], acc.at[pl.ds(r0, _ROWS)])
    plsc.subcore_barrier()
    base = wid * _EPW
    idx = (i0, i1)
    bv = (v0, v1)
    sl = (sl0, sl1)
    ss = (ss0, ss1)

    def issue_ld(ch, b):
        o = base + ch * _CHUNK
        pltpu.async_copy(rcv.at[pl.ds(o, _CHUNK)], idx[b], sl[b])
        pltpu.async_copy(vals.at[pl.ds(o, _CHUNK)], bv[b], sl[b])

    def wait_ld(ch, b):
        o = base + ch * _CHUNK
        pltpu.make_async_copy(rcv.at[pl.ds(o, _CHUNK)], idx[b], sl[b]).wait()
        pltpu.make_async_copy(vals.at[pl.ds(o, _CHUNK)], bv[b], sl[b]).wait()

    def issue_sc(b):
        pltpu.async_copy(bv[b], acc.at[idx[b]], ss[b], add=True)

    def wait_sc(b):
        pltpu.make_async_copy(bv[b], acc.at[idx[b]], ss[b]).wait()

    issue_ld(0, 0)

    def body(g, carry):
        wait_ld(2 * g, 0)
        issue_sc(0)

        @pl.when(g > 0)
        def _():
            wait_sc(1)

        issue_ld(2 * g + 1, 1)
        wait_ld(2 * g + 1, 1)
        issue_sc(1)
        wait_sc(0)

        @pl.when(g < _EPW // _CHUNK // 2 - 1)
        def _():
            issue_ld(2 * g + 2, 0)

        return carry

    lax.fori_loop(0, _EPW // _CHUNK // 2, body, 0)
    wait_sc(1)
    plsc.subcore_barrier()
    pltpu.sync_copy(acc.at[pl.ds(r0, _ROWS)], part_out.at[cid, pl.ds(r0, _ROWS)])


def _full(shape):
    return pl.BlockSpec(shape, lambda i: tuple(0 for _ in shape))


_proj_call = pl.pallas_call(
    _proj_body,
    grid=(_N_PAD // _BN,),
    in_specs=[
        pl.BlockSpec((_BN, _F), lambda i: (i, 0)),
        pl.BlockSpec((_BN, 16), lambda i: (i, 0)),
        _full((_F, _F)),
        _full((_F, 2 * _F)),
    ],
    out_specs=[
        pl.BlockSpec((_BN, _F), lambda i: (i, 0)),
        pl.BlockSpec((_BN, 2 * _F), lambda i: (i, 0)),
    ],
    out_shape=[
        jax.ShapeDtypeStruct((_N_PAD, _F), jnp.float32),
        jax.ShapeDtypeStruct((_N_PAD, 2 * _F), jnp.float32),
    ],
)

_edge_call = pl.pallas_call(
    _edge_body,
    grid=(_E_HALF // _BE,),
    in_specs=[
        pl.BlockSpec((_BE, 16), lambda i: (i, 0)),
        pl.BlockSpec((_BE, 16), lambda i: (i, 0)),
        pl.BlockSpec((_BE, _F), lambda i: (i, 0)),
        pl.BlockSpec((_BE, 2 * _F), lambda i: (i, 0)),
        _full((16, _F)), _full((1, _F)), _full((_F, _F)), _full((1, _F)),
        _full((16, _F)), _full((1, _F)), _full((_F, _F)), _full((1, _F)),
        _full((16, _F)), _full((1, _F)), _full((_F, 16)), _full((1, 16)),
        _full((16, _F)), _full((1, _F)), _full((_F, 16)), _full((1, 16)),
        _full((16, 16)), _full((16, 16)), _full((_F, _F)),
    ],
    out_specs=[
        pl.BlockSpec((_BE, _F), lambda i: (i, 0)),
        pl.BlockSpec((_BE, _F), lambda i: (i, 0)),
    ],
    out_shape=[
        jax.ShapeDtypeStruct((_E_HALF, _F), jnp.float32),
        jax.ShapeDtypeStruct((_E_HALF, _F), jnp.float32),
    ],
)

_fin_call = pl.pallas_call(
    _fin_body,
    grid=(_N_PAD // _BN,),
    in_specs=[
        pl.BlockSpec((_BN, _F), lambda i: (i, 0)),
        pl.BlockSpec((_BN, 16), lambda i: (i, 0)),
        pl.BlockSpec((2, _BN, _F), lambda i: (0, i, 0)),
        pl.BlockSpec((2, _BN, _F), lambda i: (0, i, 0)),
        pl.BlockSpec((2, _BN, _F), lambda i: (0, i, 0)),
        pl.BlockSpec((2, _BN, _F), lambda i: (0, i, 0)),
        _full((136, _F)), _full((1, _F)), _full((_F, 136)), _full((1, 136)),
        _full((16, 16)), _full((8, 16)),
    ],
    out_specs=[
        pl.BlockSpec((_BN, _F), lambda i: (i, 0)),
        pl.BlockSpec((_BN, 16), lambda i: (i, 0)),
    ],
    out_shape=[
        jax.ShapeDtypeStruct((_N_PAD, _F), jnp.float32),
        jax.ShapeDtypeStruct((_N_PAD, 16), jnp.float32),
    ],
)

_sc_calls_cache = []


def _sc_calls():
    if _sc_calls_cache:
        return _sc_calls_cache[0]
    mesh = plsc.VectorSubcoreMesh(core_axis_name="c", subcore_axis_name="s")
    gather_call = pl.kernel(
        _gather_body,
        out_type=[
            jax.ShapeDtypeStruct((_E_HALF, _F), jnp.float32),
            jax.ShapeDtypeStruct((_E_HALF, 2 * _F), jnp.float32),
        ],
        mesh=mesh,
        scratch_types=[
            pltpu.VMEM((_GCHUNK,), jnp.int32),
            pltpu.VMEM((_GCHUNK,), jnp.int32),
            pltpu.VMEM((_GCHUNK,), jnp.int32),
            pltpu.VMEM((_GCHUNK,), jnp.int32),
            pltpu.VMEM((_GCHUNK, _F), jnp.float32),
            pltpu.VMEM((_GCHUNK, _F), jnp.float32),
            pltpu.VMEM((_GCHUNK, 2 * _F), jnp.float32),
            pltpu.VMEM((_GCHUNK, 2 * _F), jnp.float32),
            pltpu.SemaphoreType.DMA,
            pltpu.SemaphoreType.DMA,
            pltpu.SemaphoreType.DMA,
            pltpu.SemaphoreType.DMA,
            pltpu.SemaphoreType.DMA,
            pltpu.SemaphoreType.DMA,
        ],
    )
    scatter_call = pl.kernel(
        _scatter_body,
        out_type=jax.ShapeDtypeStruct((2, _N_PAD, _F), jnp.float32),
        mesh=mesh,
        scratch_types=[
            pltpu.VMEM_SHARED((_N_PAD, _F), jnp.float32),
            pltpu.VMEM((_CHUNK,), jnp.int32),
            pltpu.VMEM((_CHUNK,), jnp.int32),
            pltpu.VMEM((_CHUNK, _F), jnp.float32),
            pltpu.VMEM((_CHUNK, _F), jnp.float32),
            pltpu.SemaphoreType.DMA,
            pltpu.SemaphoreType.DMA,
            pltpu.SemaphoreType.DMA,
            pltpu.SemaphoreType.DMA,
        ],
    )
    _sc_calls_cache.append((gather_call, scatter_call))
    return _sc_calls_cache[0]


def kernel(node_feats, chi, edge_feats, edge_sh, cutoffs, senders, receivers,
           W_fbr0, b_fbr0, W_fbr1, b_fbr1, W_fbs0, b_fbs0, W_fbs1, b_fbs1,
           Wq, Wk, Wv,
           W_gbr0, b_gbr0, W_gbr1, b_gbr1, W_gbs0, b_gbs0, W_gbs1, b_gbs1,
           Wi0, bi0, Wi1, bi1):
    f32 = jnp.float32

    nf_pad = jnp.pad(node_feats, ((0, _N_PAD - _N), (0, 0)))
    chi_pad = jnp.pad(chi, ((0, _N_PAD - _N), (0, 1)))
    ef_pad = jnp.pad(edge_feats, ((0, _E_PAD - _E), (0, 0)))
    shp = jnp.pad(jnp.concatenate([edge_sh, cutoffs[:, None]], axis=1),
                  ((0, _E_PAD - _E), (0, 0)))
    snd = jnp.pad(senders, (0, _E_PAD - _E))
    rcv = jnp.pad(receivers, (0, _E_PAD - _E))

    wkv = jnp.concatenate([Wk, Wv], axis=1)
    wfbs0 = jnp.zeros((16, _F), f32).at[:3].set(W_fbs0)
    wgbs0 = jnp.zeros((16, _F), f32).at[:3].set(W_gbs0)
    wgbr1 = jnp.zeros((_F, 16), f32).at[:, :3].set(W_gbr1)
    wgbs1 = jnp.zeros((_F, 16), f32).at[:, :3].set(W_gbs1)
    bgbr1 = jnp.zeros((16,), f32).at[:3].set(b_gbr1)
    bgbs1 = jnp.zeros((16,), f32).at[:3].set(b_gbs1)
    wi0 = jnp.zeros((136, _F), f32).at[:131].set(Wi0)
    wi1 = jnp.zeros((_F, 136), f32).at[:, :131].set(Wi1)
    bi1p = jnp.zeros((136,), f32).at[:131].set(bi1)

    g16 = jnp.asarray(_G16)
    m16 = jnp.asarray(_M16)
    m8 = jnp.asarray(_M8)
    hmask = jnp.asarray(_HMASK)

    gather_call, scatter_call = _sc_calls()
    qc, kvc = _proj_call(nf_pad, chi_pad, Wq, wkv)
    zeros = jnp.zeros((_N_PAD, _F), f32)

    def edge_half(lo, qcr, kvcs):
        return _edge_call(
            lax.dynamic_slice_in_dim(ef_pad, lo, _E_HALF),
            lax.dynamic_slice_in_dim(shp, lo, _E_HALF),
            qcr, kvcs,
            W_fbr0, b_fbr0[None], W_fbr1, b_fbr1[None],
            wfbs0, b_fbs0[None], W_fbs1, b_fbs1[None],
            W_gbr0, b_gbr0[None], wgbr1, bgbr1[None],
            wgbs0, b_gbs0[None], wgbs1, bgbs1[None],
            g16, m16, hmask)

    snd0, rcv0 = snd[:_E_HALF], rcv[:_E_HALF]
    snd1, rcv1 = snd[_E_HALF:], rcv[_E_HALF:]
    qcr0, kvcs0 = gather_call(qc, kvc, snd0, rcv0)
    qcr1, kvcs1 = gather_call(qc, kvc, snd1, rcv1)
    msg0, gv0 = edge_half(0, qcr0, kvcs0)
    msg1, gv1 = edge_half(_E_HALF, qcr1, kvcs1)
    pm0 = scatter_call(rcv0, msg0, zeros)
    pg0 = scatter_call(rcv0, gv0, zeros)
    pm1 = scatter_call(rcv1, msg1, zeros)
    pg1 = scatter_call(rcv1, gv1, zeros)
    outn, outc = _fin_call(nf_pad, chi_pad, pm0, pm1, pg0, pg1,
                           wi0, bi0[None], wi1, bi1p[None], g16, m8)
    return outn[:_N], outc[:_N, :_SPH]
